# baseline (speedup 1.0000x reference)
import functools

import jax
import jax.numpy as jnp
from jax import lax
from jax.experimental import pallas as pl
from jax.experimental.pallas import tpu as pltpu
from jax.experimental.pallas import tpu_sc as plsc

N = 10000
E = 320000
D = 128
NC = 2
NS = 16
NW = NC * NS
L = 16

DEBUG_SKIP_EDGES = False

CB = 80
EPT = E // NW
NCHUNK = EPT // CB
RPT = (N // NS) // 8 * 8
RTAIL = N - RPT * NS
ZB = 48


def _agg_body(with_cnt, x_hbm, src_hbm, dst_hbm, ew_hbm, z128_hbm, z16_hbm,
              ones_hbm, agg_out, cnt_out, acc_sh, cnt_sh, sidx_v, didx_v,
              rows_v, ones_v, zbuf16_v, ew_v, didx_b_v, rows_b_v, ew_b_v,
              gsem, gsemb):
    c = lax.axis_index("c")
    s = lax.axis_index("s")
    tile_base = c * (E // NC) + s * EPT

    zstage = rows_v.at[pl.ds(0, ZB)]
    pltpu.sync_copy(z128_hbm.at[pl.ds(0, ZB)], zstage)
    if with_cnt:
        pltpu.sync_copy(z16_hbm.at[pl.ds(0, ZB)], zbuf16_v)
        pltpu.sync_copy(ones_hbm, ones_v)

    def zero_body(i, carry):
        row = s * RPT + i * ZB
        pltpu.sync_copy(zstage, acc_sh.at[pl.ds(row, ZB)])
        if with_cnt:
            pltpu.sync_copy(zbuf16_v, cnt_sh.at[pl.ds(row, ZB)])
        return carry

    lax.fori_loop(0, RPT // ZB, zero_body, 0)

    @pl.when(s == NS - 1)
    def _zero_tail():
        pltpu.sync_copy(rows_v.at[pl.ds(0, RTAIL)],
                        acc_sh.at[pl.ds(RPT * NS, RTAIL)])
        if with_cnt:
            pltpu.sync_copy(zbuf16_v.at[pl.ds(0, RTAIL)],
                            cnt_sh.at[pl.ds(RPT * NS, RTAIL)])

    plsc.subcore_barrier()

    pltpu.sync_copy(src_hbm.at[pl.ds(tile_base, EPT)], sidx_v)

    def issue(j, didx_b, ew_b, rows_b, sem):
        d1 = pltpu.async_copy(dst_hbm.at[pl.ds(tile_base + j * CB, CB)],
                              didx_b, sem)
        d2 = pltpu.async_copy(ew_hbm.at[pl.ds(tile_base + j * CB, CB)],
                              ew_b, sem)
        d3 = pltpu.async_copy(x_hbm.at[sidx_v.at[pl.ds(j * CB, CB)]],
                              rows_b, sem)
        return d1, d2, d3

    def process(didx_b, ew_b, rows_b):
        @plsc.parallel_loop(0, CB, unroll=8)
        def _scale(i):
            w = plsc.load_gather(ew_b, [jnp.full((L,), i, jnp.int32)])
            for k in range(D // L):
                sl = (i, pl.ds(k * L, L))
                rows_b[sl] = rows_b[sl] * w

        pltpu.sync_copy(rows_b, acc_sh.at[didx_b], add=True)
        if with_cnt:
            pltpu.sync_copy(ones_v, cnt_sh.at[didx_b], add=True)

    def pair_body(t, carry):
        j = 2 * t
        da = issue(j, didx_v, ew_v, rows_v, gsem)
        db = issue(j + 1, didx_b_v, ew_b_v, rows_b_v, gsemb)
        for d in da:
            d.wait()
        process(didx_v, ew_v, rows_v)
        for d in db:
            d.wait()
        process(didx_b_v, ew_b_v, rows_b_v)
        return carry

    if not DEBUG_SKIP_EDGES:
        lax.fori_loop(0, NCHUNK // 2, pair_body, 0)
        if NCHUNK % 2:
            for d in issue(NCHUNK - 1, didx_v, ew_v, rows_v, gsem):
                d.wait()
            process(didx_v, ew_v, rows_v)
    plsc.subcore_barrier()

    def out_body(i, carry):
        row = s * RPT + i * ZB
        pltpu.sync_copy(acc_sh.at[pl.ds(row, ZB)], zstage)
        pltpu.sync_copy(zstage, agg_out.at[pl.ds(c * N + row, ZB)])
        if with_cnt:
            pltpu.sync_copy(cnt_sh.at[pl.ds(row, ZB)], zbuf16_v)
            pltpu.sync_copy(zbuf16_v, cnt_out.at[pl.ds(c * N + row, ZB)])
        return carry

    lax.fori_loop(0, RPT // ZB, out_body, 0)

    @pl.when(s == NS - 1)
    def _copy_tail():
        pltpu.sync_copy(acc_sh.at[pl.ds(RPT * NS, RTAIL)],
                        rows_v.at[pl.ds(0, RTAIL)])
        pltpu.sync_copy(rows_v.at[pl.ds(0, RTAIL)],
                        agg_out.at[pl.ds(c * N + RPT * NS, RTAIL)])
        if with_cnt:
            pltpu.sync_copy(cnt_sh.at[pl.ds(RPT * NS, RTAIL)],
                            zbuf16_v.at[pl.ds(0, RTAIL)])
            pltpu.sync_copy(zbuf16_v.at[pl.ds(0, RTAIL)],
                            cnt_out.at[pl.ds(c * N + RPT * NS, RTAIL)])


def _make_agg_kernel(with_cnt):
    mesh = plsc.VectorSubcoreMesh(core_axis_name="c", subcore_axis_name="s",
                                  num_cores=NC, num_subcores=NS)
    out_type = [jax.ShapeDtypeStruct((NC * N, D), jnp.float32),
                jax.ShapeDtypeStruct((NC * N, L), jnp.float32)]
    scratch = [
        pltpu.VMEM_SHARED((N, D), jnp.float32),
        pltpu.VMEM_SHARED((N, L), jnp.float32),
        pltpu.VMEM((EPT,), jnp.int32),
        pltpu.VMEM((CB,), jnp.int32),
        pltpu.VMEM((CB, D), jnp.float32),
        pltpu.VMEM((CB, L), jnp.float32),
        pltpu.VMEM((ZB, L), jnp.float32),
        pltpu.VMEM((CB,), jnp.float32),
        pltpu.VMEM((CB,), jnp.int32),
        pltpu.VMEM((CB, D), jnp.float32),
        pltpu.VMEM((CB,), jnp.float32),
        pltpu.SemaphoreType.DMA,
        pltpu.SemaphoreType.DMA,
    ]
    return pl.kernel(functools.partial(_agg_body, with_cnt), out_type,
                     mesh=mesh, scratch_types=scratch,
                     compiler_params=pltpu.CompilerParams(
                         needs_layout_passes=False,
                         use_tc_tiling_on_sc=False))


def _score_body(h_hbm, src_hbm, dst_hbm, out_hbm, sidx_v, didx_v, srows_v,
                drows_v, srows_b_v, drows_b_v, score_v, tmp_v, gsem, dsem):
    c = lax.axis_index("c")
    s = lax.axis_index("s")
    tile_base = c * (E // NC) + s * EPT
    lanes = lax.broadcasted_iota(jnp.int32, (L,), 0)

    pltpu.sync_copy(src_hbm.at[pl.ds(tile_base, EPT)], sidx_v)
    pltpu.sync_copy(dst_hbm.at[pl.ds(tile_base, EPT)], didx_v)

    def issue(j, srows_b, drows_b, sem):
        cp_s = pltpu.async_copy(h_hbm.at[sidx_v.at[pl.ds(j * CB, CB)]],
                                srows_b, sem)
        cp_d = pltpu.async_copy(h_hbm.at[didx_v.at[pl.ds(j * CB, CB)]],
                                drows_b, sem)
        return cp_s, cp_d

    def process(j, srows_v, drows_v):
        base = tile_base + j * CB

        @plsc.parallel_loop(0, CB // L, unroll=2)
        def _group(g):
            gv = jnp.full((L,), g, jnp.int32)
            for e in range(L):
                i = g * L + e
                ps = []
                for k in range(D // L):
                    sl = (i, pl.ds(k * L, L))
                    ps.append(srows_v[sl] * drows_v[sl])
                acc = ((ps[0] + ps[1]) + (ps[2] + ps[3])) + \
                      ((ps[4] + ps[5]) + (ps[6] + ps[7]))
                tmp_v[g, e, pl.ds(0, L)] = acc
            tots = [jnp.zeros((L,), jnp.float32) for _ in range(4)]
            for j in range(L):
                jv = jnp.full((L,), j, jnp.int32)
                tots[j % 4] = tots[j % 4] + plsc.load_gather(
                    tmp_v, [gv, lanes, jv])
            score_v[pl.ds(g * L, L)] = ((tots[0] + tots[1])
                                        + (tots[2] + tots[3]))

        pltpu.sync_copy(score_v, out_hbm.at[pl.ds(base, CB)])

    def pair_body(t, carry):
        j = 2 * t
        da = issue(j, srows_v, drows_v, gsem)
        db = issue(j + 1, srows_b_v, drows_b_v, dsem)
        for d in da:
            d.wait()
        process(j, srows_v, drows_v)
        for d in db:
            d.wait()
        process(j + 1, srows_b_v, drows_b_v)
        return carry

    lax.fori_loop(0, NCHUNK // 2, pair_body, 0)
    if NCHUNK % 2:
        for d in issue(NCHUNK - 1, srows_v, drows_v, gsem):
            d.wait()
        process(NCHUNK - 1, srows_v, drows_v)


def _make_score_kernel():
    mesh = plsc.VectorSubcoreMesh(core_axis_name="c", subcore_axis_name="s",
                                  num_cores=NC, num_subcores=NS)
    out_type = jax.ShapeDtypeStruct((E,), jnp.float32)
    scratch = [
        pltpu.VMEM((EPT,), jnp.int32),
        pltpu.VMEM((EPT,), jnp.int32),
        pltpu.VMEM((CB, D), jnp.float32),
        pltpu.VMEM((CB, D), jnp.float32),
        pltpu.VMEM((CB, D), jnp.float32),
        pltpu.VMEM((CB, D), jnp.float32),
        pltpu.VMEM((CB,), jnp.float32),
        pltpu.VMEM((CB // L, L, 17), jnp.float32),
        pltpu.SemaphoreType.DMA,
        pltpu.SemaphoreType.DMA,
    ]
    return pl.kernel(_score_body, out_type, mesh=mesh, scratch_types=scratch,
                     compiler_params=pltpu.CompilerParams(
                         needs_layout_passes=False,
                         use_tc_tiling_on_sc=False))


def _layer_tc_body(apply_sigmoid, agg_a_ref, agg_b_ref, cnt_a_ref, cnt_b_ref,
                   x_ref, wrel_ref, brel_ref, wroot_ref, o_ref):
    aggsum = agg_a_ref[...] + agg_b_ref[...]
    cnt = cnt_a_ref[:, :1] + cnt_b_ref[:, :1]
    inv = 1.0 / jnp.maximum(cnt, 1.0)
    h = (jnp.dot(aggsum * inv, wrel_ref[...].T,
                 preferred_element_type=jnp.float32)
         + brel_ref[...]
         + jnp.dot(x_ref[...], wroot_ref[...].T,
                   preferred_element_type=jnp.float32))
    if apply_sigmoid:
        h = jax.nn.sigmoid(h)
    o_ref[...] = h


def _layer_tc(agg, cnt, x, wrel, brel, wroot, apply_sigmoid, block=1000):
    grid = N // block
    return pl.pallas_call(
        functools.partial(_layer_tc_body, apply_sigmoid),
        grid=(grid,),
        in_specs=[
            pl.BlockSpec((block, D), lambda i: (i, 0)),
            pl.BlockSpec((block, D), lambda i: (i, 0)),
            pl.BlockSpec((block, L), lambda i: (i, 0)),
            pl.BlockSpec((block, L), lambda i: (i, 0)),
            pl.BlockSpec((block, D), lambda i: (i, 0)),
            pl.BlockSpec((D, D), lambda i: (0, 0)),
            pl.BlockSpec((1, D), lambda i: (0, 0)),
            pl.BlockSpec((D, D), lambda i: (0, 0)),
        ],
        out_specs=pl.BlockSpec((block, D), lambda i: (i, 0)),
        out_shape=jax.ShapeDtypeStruct((N, D), jnp.float32),
    )(agg[:N], agg[N:], cnt[:N], cnt[N:], x, wrel, brel, wroot)


def kernel(x, edge_index, edge_attr, Wrel1, brel1, Wroot1, Wrel2, brel2,
           Wroot2):
    src = edge_index[0]
    dst = edge_index[1]
    z128 = jnp.zeros((N, D), jnp.float32)
    z16 = jnp.zeros((N, L), jnp.float32)
    ones = jnp.ones((CB, L), jnp.float32)

    agg_with_cnt = _make_agg_kernel(True)
    agg_no_cnt = _make_agg_kernel(False)
    score_k = _make_score_kernel()

    agg1, cnt = agg_with_cnt(x, src, dst, edge_attr, z128, z16, ones)
    h1 = _layer_tc(agg1, cnt, x, Wrel1, brel1.reshape(1, D), Wroot1, True)
    agg2, _ = agg_no_cnt(h1, src, dst, edge_attr, z128, z16, ones)
    h2 = _layer_tc(agg2, cnt, h1, Wrel2, brel2.reshape(1, D), Wroot2, False)
    return score_k(h2, src, dst)


# speedup vs baseline: 1.0800x; 1.0800x over previous
import functools

import jax
import jax.numpy as jnp
from jax import lax
from jax.experimental import pallas as pl
from jax.experimental.pallas import tpu as pltpu
from jax.experimental.pallas import tpu_sc as plsc

N = 10000
E = 320000
D = 128
NC = 2
NS = 16
NW = NC * NS
L = 16

DEBUG_SKIP_EDGES = False

CB = 80
EPT = E // NW
NCHUNK = EPT // CB
RPT = (N // NS) // 8 * 8
RTAIL = N - RPT * NS
ZB = 48


def _agg_body(with_cnt, x_hbm, src_hbm, dst_hbm, ew_hbm, z128_hbm, z16_hbm,
              ones_hbm, agg_out, cnt_out, acc_sh, cnt_sh, sidx_v, didx_v,
              rows_v, ones_v, zbuf16_v, ew_v, didx_b_v, rows_b_v, ew_b_v,
              gsem, gsemb):
    c = lax.axis_index("c")
    s = lax.axis_index("s")
    tile_base = c * (E // NC) + s * EPT

    zstage = rows_v.at[pl.ds(0, ZB)]
    pltpu.sync_copy(z128_hbm.at[pl.ds(0, ZB)], zstage)
    if with_cnt:
        pltpu.sync_copy(z16_hbm.at[pl.ds(0, ZB)], zbuf16_v)
        pltpu.sync_copy(ones_hbm, ones_v)

    def zero_body(i, carry):
        row = s * RPT + i * ZB
        pltpu.sync_copy(zstage, acc_sh.at[pl.ds(row, ZB)])
        if with_cnt:
            pltpu.sync_copy(zbuf16_v, cnt_sh.at[pl.ds(row, ZB)])
        return carry

    lax.fori_loop(0, RPT // ZB, zero_body, 0)

    @pl.when(s == NS - 1)
    def _zero_tail():
        pltpu.sync_copy(rows_v.at[pl.ds(0, RTAIL)],
                        acc_sh.at[pl.ds(RPT * NS, RTAIL)])
        if with_cnt:
            pltpu.sync_copy(zbuf16_v.at[pl.ds(0, RTAIL)],
                            cnt_sh.at[pl.ds(RPT * NS, RTAIL)])

    plsc.subcore_barrier()

    pltpu.sync_copy(src_hbm.at[pl.ds(tile_base, EPT)], sidx_v)

    def issue(j, didx_b, ew_b, rows_b, sem):
        d1 = pltpu.async_copy(dst_hbm.at[pl.ds(tile_base + j * CB, CB)],
                              didx_b, sem)
        d2 = pltpu.async_copy(ew_hbm.at[pl.ds(tile_base + j * CB, CB)],
                              ew_b, sem)
        d3 = pltpu.async_copy(x_hbm.at[sidx_v.at[pl.ds(j * CB, CB)]],
                              rows_b, sem)
        return d1, d2, d3

    def process(didx_b, ew_b, rows_b):
        @plsc.parallel_loop(0, CB, unroll=4)
        def _scale(i):
            w = plsc.load_gather(ew_b, [jnp.full((L,), i, jnp.int32)])
            for k in range(D // L):
                sl = (i, pl.ds(k * L, L))
                rows_b[sl] = rows_b[sl] * w

        pltpu.sync_copy(rows_b, acc_sh.at[didx_b], add=True)
        if with_cnt:
            pltpu.sync_copy(ones_v, cnt_sh.at[didx_b], add=True)

    def pair_body(t, carry):
        j = 2 * t
        da = issue(j, didx_v, ew_v, rows_v, gsem)
        db = issue(j + 1, didx_b_v, ew_b_v, rows_b_v, gsemb)
        for d in da:
            d.wait()
        process(didx_v, ew_v, rows_v)
        for d in db:
            d.wait()
        process(didx_b_v, ew_b_v, rows_b_v)
        return carry

    if not DEBUG_SKIP_EDGES:
        lax.fori_loop(0, NCHUNK // 2, pair_body, 0)
        if NCHUNK % 2:
            for d in issue(NCHUNK - 1, didx_v, ew_v, rows_v, gsem):
                d.wait()
            process(didx_v, ew_v, rows_v)
    plsc.subcore_barrier()

    def out_body(i, carry):
        row = s * RPT + i * ZB
        pltpu.sync_copy(acc_sh.at[pl.ds(row, ZB)], zstage)
        pltpu.sync_copy(zstage, agg_out.at[pl.ds(c * N + row, ZB)])
        if with_cnt:
            pltpu.sync_copy(cnt_sh.at[pl.ds(row, ZB)], zbuf16_v)
            pltpu.sync_copy(zbuf16_v, cnt_out.at[pl.ds(c * N + row, ZB)])
        return carry

    lax.fori_loop(0, RPT // ZB, out_body, 0)

    @pl.when(s == NS - 1)
    def _copy_tail():
        pltpu.sync_copy(acc_sh.at[pl.ds(RPT * NS, RTAIL)],
                        rows_v.at[pl.ds(0, RTAIL)])
        pltpu.sync_copy(rows_v.at[pl.ds(0, RTAIL)],
                        agg_out.at[pl.ds(c * N + RPT * NS, RTAIL)])
        if with_cnt:
            pltpu.sync_copy(cnt_sh.at[pl.ds(RPT * NS, RTAIL)],
                            zbuf16_v.at[pl.ds(0, RTAIL)])
            pltpu.sync_copy(zbuf16_v.at[pl.ds(0, RTAIL)],
                            cnt_out.at[pl.ds(c * N + RPT * NS, RTAIL)])


def _make_agg_kernel(with_cnt):
    mesh = plsc.VectorSubcoreMesh(core_axis_name="c", subcore_axis_name="s",
                                  num_cores=NC, num_subcores=NS)
    out_type = [jax.ShapeDtypeStruct((NC * N, D), jnp.float32),
                jax.ShapeDtypeStruct((NC * N, L), jnp.float32)]
    scratch = [
        pltpu.VMEM_SHARED((N, D), jnp.float32),
        pltpu.VMEM_SHARED((N, L), jnp.float32),
        pltpu.VMEM((EPT,), jnp.int32),
        pltpu.VMEM((CB,), jnp.int32),
        pltpu.VMEM((CB, D), jnp.float32),
        pltpu.VMEM((CB, L), jnp.float32),
        pltpu.VMEM((ZB, L), jnp.float32),
        pltpu.VMEM((CB,), jnp.float32),
        pltpu.VMEM((CB,), jnp.int32),
        pltpu.VMEM((CB, D), jnp.float32),
        pltpu.VMEM((CB,), jnp.float32),
        pltpu.SemaphoreType.DMA,
        pltpu.SemaphoreType.DMA,
    ]
    return pl.kernel(functools.partial(_agg_body, with_cnt), out_type,
                     mesh=mesh, scratch_types=scratch,
                     compiler_params=pltpu.CompilerParams(
                         needs_layout_passes=False,
                         use_tc_tiling_on_sc=False))


def _score_body(h_hbm, src_hbm, dst_hbm, out_hbm, sidx_v, didx_v, srows_v,
                drows_v, srows_b_v, drows_b_v, score_v, tmp_v, gsem, dsem):
    c = lax.axis_index("c")
    s = lax.axis_index("s")
    tile_base = c * (E // NC) + s * EPT
    lanes = lax.broadcasted_iota(jnp.int32, (L,), 0)

    pltpu.sync_copy(src_hbm.at[pl.ds(tile_base, EPT)], sidx_v)
    pltpu.sync_copy(dst_hbm.at[pl.ds(tile_base, EPT)], didx_v)

    def issue(j, srows_b, drows_b, sem):
        cp_s = pltpu.async_copy(h_hbm.at[sidx_v.at[pl.ds(j * CB, CB)]],
                                srows_b, sem)
        cp_d = pltpu.async_copy(h_hbm.at[didx_v.at[pl.ds(j * CB, CB)]],
                                drows_b, sem)
        return cp_s, cp_d

    def process(j, srows_v, drows_v):
        base = tile_base + j * CB

        @plsc.parallel_loop(0, CB // L, unroll=1)
        def _group(g):
            gv = jnp.full((L,), g, jnp.int32)
            for e in range(L):
                i = g * L + e
                ps = []
                for k in range(D // L):
                    sl = (i, pl.ds(k * L, L))
                    ps.append(srows_v[sl] * drows_v[sl])
                acc = ((ps[0] + ps[1]) + (ps[2] + ps[3])) + \
                      ((ps[4] + ps[5]) + (ps[6] + ps[7]))
                tmp_v[g, e, pl.ds(0, L)] = acc
            tots = [jnp.zeros((L,), jnp.float32) for _ in range(4)]
            for j in range(L):
                jv = jnp.full((L,), j, jnp.int32)
                tots[j % 4] = tots[j % 4] + plsc.load_gather(
                    tmp_v, [gv, lanes, jv])
            score_v[pl.ds(g * L, L)] = ((tots[0] + tots[1])
                                        + (tots[2] + tots[3]))

        pltpu.sync_copy(score_v, out_hbm.at[pl.ds(base, CB)])

    def pair_body(t, carry):
        j = 2 * t
        da = issue(j, srows_v, drows_v, gsem)
        db = issue(j + 1, srows_b_v, drows_b_v, dsem)
        for d in da:
            d.wait()
        process(j, srows_v, drows_v)
        for d in db:
            d.wait()
        process(j + 1, srows_b_v, drows_b_v)
        return carry

    lax.fori_loop(0, NCHUNK // 2, pair_body, 0)
    if NCHUNK % 2:
        for d in issue(NCHUNK - 1, srows_v, drows_v, gsem):
            d.wait()
        process(NCHUNK - 1, srows_v, drows_v)


def _make_score_kernel():
    mesh = plsc.VectorSubcoreMesh(core_axis_name="c", subcore_axis_name="s",
                                  num_cores=NC, num_subcores=NS)
    out_type = jax.ShapeDtypeStruct((E,), jnp.float32)
    scratch = [
        pltpu.VMEM((EPT,), jnp.int32),
        pltpu.VMEM((EPT,), jnp.int32),
        pltpu.VMEM((CB, D), jnp.float32),
        pltpu.VMEM((CB, D), jnp.float32),
        pltpu.VMEM((CB, D), jnp.float32),
        pltpu.VMEM((CB, D), jnp.float32),
        pltpu.VMEM((CB,), jnp.float32),
        pltpu.VMEM((CB // L, L, 17), jnp.float32),
        pltpu.SemaphoreType.DMA,
        pltpu.SemaphoreType.DMA,
    ]
    return pl.kernel(_score_body, out_type, mesh=mesh, scratch_types=scratch,
                     compiler_params=pltpu.CompilerParams(
                         needs_layout_passes=False,
                         use_tc_tiling_on_sc=False))


def _layer_tc_body(apply_sigmoid, agg_a_ref, agg_b_ref, cnt_a_ref, cnt_b_ref,
                   x_ref, wrel_ref, brel_ref, wroot_ref, o_ref):
    aggsum = agg_a_ref[...] + agg_b_ref[...]
    cnt = cnt_a_ref[:, :1] + cnt_b_ref[:, :1]
    inv = 1.0 / jnp.maximum(cnt, 1.0)
    h = (jnp.dot(aggsum * inv, wrel_ref[...].T,
                 preferred_element_type=jnp.float32)
         + brel_ref[...]
         + jnp.dot(x_ref[...], wroot_ref[...].T,
                   preferred_element_type=jnp.float32))
    if apply_sigmoid:
        h = jax.nn.sigmoid(h)
    o_ref[...] = h


def _layer_tc(agg, cnt, x, wrel, brel, wroot, apply_sigmoid, block=1000):
    grid = N // block
    return pl.pallas_call(
        functools.partial(_layer_tc_body, apply_sigmoid),
        grid=(grid,),
        in_specs=[
            pl.BlockSpec((block, D), lambda i: (i, 0)),
            pl.BlockSpec((block, D), lambda i: (i, 0)),
            pl.BlockSpec((block, L), lambda i: (i, 0)),
            pl.BlockSpec((block, L), lambda i: (i, 0)),
            pl.BlockSpec((block, D), lambda i: (i, 0)),
            pl.BlockSpec((D, D), lambda i: (0, 0)),
            pl.BlockSpec((1, D), lambda i: (0, 0)),
            pl.BlockSpec((D, D), lambda i: (0, 0)),
        ],
        out_specs=pl.BlockSpec((block, D), lambda i: (i, 0)),
        out_shape=jax.ShapeDtypeStruct((N, D), jnp.float32),
    )(agg[:N], agg[N:], cnt[:N], cnt[N:], x, wrel, brel, wroot)


def kernel(x, edge_index, edge_attr, Wrel1, brel1, Wroot1, Wrel2, brel2,
           Wroot2):
    src = edge_index[0]
    dst = edge_index[1]
    z128 = jnp.zeros((N, D), jnp.float32)
    z16 = jnp.zeros((N, L), jnp.float32)
    ones = jnp.ones((CB, L), jnp.float32)

    agg_with_cnt = _make_agg_kernel(True)
    agg_no_cnt = _make_agg_kernel(False)
    score_k = _make_score_kernel()

    agg1, cnt = agg_with_cnt(x, src, dst, edge_attr, z128, z16, ones)
    h1 = _layer_tc(agg1, cnt, x, Wrel1, brel1.reshape(1, D), Wroot1, True)
    agg2, _ = agg_no_cnt(h1, src, dst, edge_attr, z128, z16, ones)
    h2 = _layer_tc(agg2, cnt, h1, Wrel2, brel2.reshape(1, D), Wroot2, False)
    return score_k(h2, src, dst)


# speedup vs baseline: 1.0808x; 1.0008x over previous
import functools

import jax
import jax.numpy as jnp
from jax import lax
from jax.experimental import pallas as pl
from jax.experimental.pallas import tpu as pltpu
from jax.experimental.pallas import tpu_sc as plsc

N = 10000
E = 320000
D = 128
NC = 2
NS = 16
NW = NC * NS
L = 16

CB = 80
EPT = E // NW
NCHUNK = EPT // CB
RPT = (N // NS) // 8 * 8
RTAIL = N - RPT * NS
ZB = 48


def _agg_body(with_cnt, x_hbm, src_hbm, dst_hbm, ew_hbm, z128_hbm, z16_hbm,
              ones_hbm, agg_out, cnt_out, acc_sh, cnt_sh, sidx_v, didx_v,
              rows_v, ones_v, zbuf16_v, ew_v, didx_b_v, rows_b_v, ew_b_v,
              gsem, gsemb):
    c = lax.axis_index("c")
    s = lax.axis_index("s")
    tile_base = c * (E // NC) + s * EPT

    zstage = rows_v.at[pl.ds(0, ZB)]
    pltpu.sync_copy(z128_hbm.at[pl.ds(0, ZB)], zstage)
    if with_cnt:
        pltpu.sync_copy(z16_hbm.at[pl.ds(0, ZB)], zbuf16_v)
        pltpu.sync_copy(ones_hbm, ones_v)

    def zero_body(i, carry):
        row = s * RPT + i * ZB
        pltpu.sync_copy(zstage, acc_sh.at[pl.ds(row, ZB)])
        if with_cnt:
            pltpu.sync_copy(zbuf16_v, cnt_sh.at[pl.ds(row, ZB)])
        return carry

    lax.fori_loop(0, RPT // ZB, zero_body, 0)

    @pl.when(s == NS - 1)
    def _zero_tail():
        pltpu.sync_copy(rows_v.at[pl.ds(0, RTAIL)],
                        acc_sh.at[pl.ds(RPT * NS, RTAIL)])
        if with_cnt:
            pltpu.sync_copy(zbuf16_v.at[pl.ds(0, RTAIL)],
                            cnt_sh.at[pl.ds(RPT * NS, RTAIL)])

    plsc.subcore_barrier()

    pltpu.sync_copy(src_hbm.at[pl.ds(tile_base, EPT)], sidx_v)

    def issue(j, didx_b, ew_b, rows_b, sem):
        d1 = pltpu.async_copy(dst_hbm.at[pl.ds(tile_base + j * CB, CB)],
                              didx_b, sem)
        d2 = pltpu.async_copy(ew_hbm.at[pl.ds(tile_base + j * CB, CB)],
                              ew_b, sem)
        d3 = pltpu.async_copy(x_hbm.at[sidx_v.at[pl.ds(j * CB, CB)]],
                              rows_b, sem)
        return d1, d2, d3

    def process(didx_b, ew_b, rows_b):
        @plsc.parallel_loop(0, CB, unroll=4)
        def _scale(i):
            w = plsc.load_gather(ew_b, [jnp.full((L,), i, jnp.int32)])
            for k in range(D // L):
                sl = (i, pl.ds(k * L, L))
                rows_b[sl] = rows_b[sl] * w

        pltpu.sync_copy(rows_b, acc_sh.at[didx_b], add=True)
        if with_cnt:
            pltpu.sync_copy(ones_v, cnt_sh.at[didx_b], add=True)

    def pair_body(t, carry):
        j = 2 * t
        da = issue(j, didx_v, ew_v, rows_v, gsem)
        db = issue(j + 1, didx_b_v, ew_b_v, rows_b_v, gsemb)
        for d in da:
            d.wait()
        process(didx_v, ew_v, rows_v)
        for d in db:
            d.wait()
        process(didx_b_v, ew_b_v, rows_b_v)
        return carry

    lax.fori_loop(0, NCHUNK // 2, pair_body, 0)
    if NCHUNK % 2:
        for d in issue(NCHUNK - 1, didx_v, ew_v, rows_v, gsem):
            d.wait()
        process(didx_v, ew_v, rows_v)
    plsc.subcore_barrier()

    def out_body(i, carry):
        row = s * RPT + i * ZB
        pltpu.sync_copy(acc_sh.at[pl.ds(row, ZB)], zstage)
        pltpu.sync_copy(zstage, agg_out.at[pl.ds(c * N + row, ZB)])
        if with_cnt:
            pltpu.sync_copy(cnt_sh.at[pl.ds(row, ZB)], zbuf16_v)
            pltpu.sync_copy(zbuf16_v, cnt_out.at[pl.ds(c * N + row, ZB)])
        return carry

    lax.fori_loop(0, RPT // ZB, out_body, 0)

    @pl.when(s == NS - 1)
    def _copy_tail():
        pltpu.sync_copy(acc_sh.at[pl.ds(RPT * NS, RTAIL)],
                        rows_v.at[pl.ds(0, RTAIL)])
        pltpu.sync_copy(rows_v.at[pl.ds(0, RTAIL)],
                        agg_out.at[pl.ds(c * N + RPT * NS, RTAIL)])
        if with_cnt:
            pltpu.sync_copy(cnt_sh.at[pl.ds(RPT * NS, RTAIL)],
                            zbuf16_v.at[pl.ds(0, RTAIL)])
            pltpu.sync_copy(zbuf16_v.at[pl.ds(0, RTAIL)],
                            cnt_out.at[pl.ds(c * N + RPT * NS, RTAIL)])


def _make_agg_kernel(with_cnt):
    mesh = plsc.VectorSubcoreMesh(core_axis_name="c", subcore_axis_name="s",
                                  num_cores=NC, num_subcores=NS)
    out_type = [jax.ShapeDtypeStruct((NC * N, D), jnp.float32),
                jax.ShapeDtypeStruct((NC * N, L), jnp.float32)]
    scratch = [
        pltpu.VMEM_SHARED((N, D), jnp.float32),
        pltpu.VMEM_SHARED((N, L), jnp.float32),
        pltpu.VMEM((EPT,), jnp.int32),
        pltpu.VMEM((CB,), jnp.int32),
        pltpu.VMEM((CB, D), jnp.float32),
        pltpu.VMEM((CB, L), jnp.float32),
        pltpu.VMEM((ZB, L), jnp.float32),
        pltpu.VMEM((CB,), jnp.float32),
        pltpu.VMEM((CB,), jnp.int32),
        pltpu.VMEM((CB, D), jnp.float32),
        pltpu.VMEM((CB,), jnp.float32),
        pltpu.SemaphoreType.DMA,
        pltpu.SemaphoreType.DMA,
    ]
    return pl.kernel(functools.partial(_agg_body, with_cnt), out_type,
                     mesh=mesh, scratch_types=scratch,
                     compiler_params=pltpu.CompilerParams(
                         needs_layout_passes=False,
                         use_tc_tiling_on_sc=False))


def _score_body(h_hbm, src_hbm, dst_hbm, out_hbm, sidx_v, didx_v, srows_v,
                drows_v, srows_b_v, drows_b_v, score_v, tmp_v, gsem, dsem):
    c = lax.axis_index("c")
    s = lax.axis_index("s")
    tile_base = c * (E // NC) + s * EPT
    lanes = lax.broadcasted_iota(jnp.int32, (L,), 0)

    pltpu.sync_copy(src_hbm.at[pl.ds(tile_base, EPT)], sidx_v)
    pltpu.sync_copy(dst_hbm.at[pl.ds(tile_base, EPT)], didx_v)

    def issue(j, srows_b, drows_b, sem):
        cp_s = pltpu.async_copy(h_hbm.at[sidx_v.at[pl.ds(j * CB, CB)]],
                                srows_b, sem)
        cp_d = pltpu.async_copy(h_hbm.at[didx_v.at[pl.ds(j * CB, CB)]],
                                drows_b, sem)
        return cp_s, cp_d

    def process(j, srows_v, drows_v):
        base = tile_base + j * CB

        @plsc.parallel_loop(0, CB // L, unroll=1)
        def _group(g):
            gv = jnp.full((L,), g, jnp.int32)
            for e in range(L):
                i = g * L + e
                ps = []
                for k in range(D // L):
                    sl = (i, pl.ds(k * L, L))
                    ps.append(srows_v[sl] * drows_v[sl])
                acc = ((ps[0] + ps[1]) + (ps[2] + ps[3])) + \
                      ((ps[4] + ps[5]) + (ps[6] + ps[7]))
                tmp_v[g, e, pl.ds(0, L)] = acc
            tots = [jnp.zeros((L,), jnp.float32) for _ in range(4)]
            for j in range(L):
                jv = jnp.full((L,), j, jnp.int32)
                tots[j % 4] = tots[j % 4] + plsc.load_gather(
                    tmp_v, [gv, lanes, jv])
            score_v[pl.ds(g * L, L)] = ((tots[0] + tots[1])
                                        + (tots[2] + tots[3]))

        pltpu.sync_copy(score_v, out_hbm.at[pl.ds(base, CB)])

    def pair_body(t, carry):
        j = 2 * t
        da = issue(j, srows_v, drows_v, gsem)
        db = issue(j + 1, srows_b_v, drows_b_v, dsem)
        for d in da:
            d.wait()
        process(j, srows_v, drows_v)
        for d in db:
            d.wait()
        process(j + 1, srows_b_v, drows_b_v)
        return carry

    lax.fori_loop(0, NCHUNK // 2, pair_body, 0)
    if NCHUNK % 2:
        for d in issue(NCHUNK - 1, srows_v, drows_v, gsem):
            d.wait()
        process(NCHUNK - 1, srows_v, drows_v)


def _make_score_kernel():
    mesh = plsc.VectorSubcoreMesh(core_axis_name="c", subcore_axis_name="s",
                                  num_cores=NC, num_subcores=NS)
    out_type = jax.ShapeDtypeStruct((E,), jnp.float32)
    scratch = [
        pltpu.VMEM((EPT,), jnp.int32),
        pltpu.VMEM((EPT,), jnp.int32),
        pltpu.VMEM((CB, D), jnp.float32),
        pltpu.VMEM((CB, D), jnp.float32),
        pltpu.VMEM((CB, D), jnp.float32),
        pltpu.VMEM((CB, D), jnp.float32),
        pltpu.VMEM((CB,), jnp.float32),
        pltpu.VMEM((CB // L, L, 17), jnp.float32),
        pltpu.SemaphoreType.DMA,
        pltpu.SemaphoreType.DMA,
    ]
    return pl.kernel(_score_body, out_type, mesh=mesh, scratch_types=scratch,
                     compiler_params=pltpu.CompilerParams(
                         needs_layout_passes=False,
                         use_tc_tiling_on_sc=False))


def _layer_tc_body(apply_sigmoid, agg_a_ref, agg_b_ref, cnt_a_ref, cnt_b_ref,
                   x_ref, wrel_ref, brel_ref, wroot_ref, o_ref):
    aggsum = agg_a_ref[...] + agg_b_ref[...]
    cnt = cnt_a_ref[:, :1] + cnt_b_ref[:, :1]
    inv = 1.0 / jnp.maximum(cnt, 1.0)
    h = (jnp.dot(aggsum * inv, wrel_ref[...].T,
                 preferred_element_type=jnp.float32)
         + brel_ref[...]
         + jnp.dot(x_ref[...], wroot_ref[...].T,
                   preferred_element_type=jnp.float32))
    if apply_sigmoid:
        h = jax.nn.sigmoid(h)
    o_ref[...] = h


def _layer_tc(agg, cnt, x, wrel, brel, wroot, apply_sigmoid, block=1000):
    grid = N // block
    return pl.pallas_call(
        functools.partial(_layer_tc_body, apply_sigmoid),
        grid=(grid,),
        in_specs=[
            pl.BlockSpec((block, D), lambda i: (i, 0)),
            pl.BlockSpec((block, D), lambda i: (i, 0)),
            pl.BlockSpec((block, L), lambda i: (i, 0)),
            pl.BlockSpec((block, L), lambda i: (i, 0)),
            pl.BlockSpec((block, D), lambda i: (i, 0)),
            pl.BlockSpec((D, D), lambda i: (0, 0)),
            pl.BlockSpec((1, D), lambda i: (0, 0)),
            pl.BlockSpec((D, D), lambda i: (0, 0)),
        ],
        out_specs=pl.BlockSpec((block, D), lambda i: (i, 0)),
        out_shape=jax.ShapeDtypeStruct((N, D), jnp.float32),
    )(agg[:N], agg[N:], cnt[:N], cnt[N:], x, wrel, brel, wroot)


def kernel(x, edge_index, edge_attr, Wrel1, brel1, Wroot1, Wrel2, brel2,
           Wroot2):
    src = edge_index[0]
    dst = edge_index[1]
    z128 = jnp.zeros((N, D), jnp.float32)
    z16 = jnp.zeros((N, L), jnp.float32)
    ones = jnp.ones((CB, L), jnp.float32)

    agg_with_cnt = _make_agg_kernel(True)
    agg_no_cnt = _make_agg_kernel(False)
    score_k = _make_score_kernel()

    agg1, cnt = agg_with_cnt(x, src, dst, edge_attr, z128, z16, ones)
    h1 = _layer_tc(agg1, cnt, x, Wrel1, brel1.reshape(1, D), Wroot1, True)
    agg2, _ = agg_no_cnt(h1, src, dst, edge_attr, z128, z16, ones)
    h2 = _layer_tc(agg2, cnt, h1, Wrel2, brel2.reshape(1, D), Wroot2, False)
    return score_k(h2, src, dst)
